# Initial kernel scaffold; baseline (speedup 1.0000x reference)
import functools

import jax
import jax.numpy as jnp
from jax import lax
from jax.experimental import pallas as pl
from jax.experimental.pallas import tpu as pltpu
from jax.experimental.pallas import tpu_sc as plsc

_NC = 2
_NS = 16
_NW = _NC * _NS
_LANES = 16
_K = 128
_ZR = 64


def _sc_mesh():
    return plsc.VectorSubcoreMesh(core_axis_name="c", subcore_axis_name="s")


def _degree_partials(idx3, n_pad):
    ew = idx3.shape[2]

    def body(idx_hbm, out_hbm, idx_v, hist_v):
        c = lax.axis_index("c")
        s = lax.axis_index("s")
        wid = c * _NS + s
        zeros = jnp.zeros((_LANES,), jnp.float32)
        ones = jnp.ones((_LANES,), jnp.float32)
        for t in range(2):
            @pl.loop(0, n_pad // _LANES)
            def _(i):
                hist_v[pl.ds(i * _LANES, _LANES)] = zeros

            pltpu.sync_copy(idx_hbm.at[t, wid], idx_v)

            @pl.loop(0, ew // _LANES)
            def _(i):
                idx = idx_v[pl.ds(i * _LANES, _LANES)]
                plsc.addupdate_scatter(hist_v, [idx], ones)

            pltpu.sync_copy(hist_v, out_hbm.at[t, wid])

    kern = pl.kernel(
        body,
        out_type=jax.ShapeDtypeStruct((2, _NW, n_pad), jnp.float32),
        mesh=_sc_mesh(),
        scratch_types=[
            pltpu.VMEM((ew,), jnp.int32),
            pltpu.VMEM((n_pad,), jnp.float32),
        ],
    )
    return kern(idx3)


def _edge_pass(h, src3, dst3, n_pad, d):
    n_chunks = src3.shape[1]
    rpw = n_pad // _NS

    def body(h_hbm, src_hbm, dst_hbm, out_hbm, sidx_v, didx_v, msg_v,
             zbuf_v, acc_sh, sem):
        c = lax.axis_index("c")
        s = lax.axis_index("s")
        wid = c * _NS + s
        zeros = jnp.zeros((_LANES,), jnp.float32)

        @pl.loop(0, _ZR)
        def _(r):
            @pl.loop(0, d // _LANES)
            def _(q):
                zbuf_v[r, pl.ds(q * _LANES, _LANES)] = zeros

        @pl.loop(0, rpw // _ZR)
        def _(i):
            pltpu.sync_copy(zbuf_v, acc_sh.at[pl.ds(s * rpw + i * _ZR, _ZR)])

        plsc.subcore_barrier()

        pltpu.sync_copy(src_hbm.at[wid], sidx_v)
        pltpu.sync_copy(dst_hbm.at[wid], didx_v)

        @pl.loop(0, n_chunks)
        def _(j):
            pltpu.async_copy(h_hbm.at[sidx_v.at[j]], msg_v, sem).wait()
            pltpu.sync_copy(msg_v, acc_sh.at[didx_v.at[j]], add=True)

        plsc.subcore_barrier()

        pltpu.sync_copy(acc_sh.at[pl.ds(s * rpw, rpw)],
                        out_hbm.at[c].at[pl.ds(s * rpw, rpw)])

    kern = pl.kernel(
        body,
        out_type=jax.ShapeDtypeStruct((_NC, n_pad, d), jnp.float32),
        mesh=_sc_mesh(),
        scratch_types=[
            pltpu.VMEM((n_chunks, _K), jnp.int32),
            pltpu.VMEM((n_chunks, _K), jnp.int32),
            pltpu.VMEM((_K, d), jnp.float32),
            pltpu.VMEM((_ZR, d), jnp.float32),
            pltpu.VMEM_SHARED((n_pad, d), jnp.float32),
            pltpu.SemaphoreType.DMA,
        ],
    )
    return kern(h, src3, dst3)


def _tc_pre(degp_ref, x_ref, w1_ref, h1_ref, sin_ref, sout_ref):
    deg = jnp.sum(degp_ref[...], axis=1)
    sout = lax.rsqrt(jnp.maximum(deg[0], 1.0))
    sin = lax.rsqrt(jnp.maximum(deg[1], 1.0))
    h = x_ref[...] * sout[:, None]
    h1_ref[...] = jnp.dot(h, w1_ref[...], preferred_element_type=jnp.float32,
                          precision=lax.Precision.HIGHEST)
    sin_ref[...] = sin[:, None]
    sout_ref[...] = sout[:, None]


def _tc_mid(p_ref, sin_ref, sout_ref, b1_ref, w2_ref, h2_ref):
    p = p_ref[...]
    agg = (p[0] + p[1]) * sin_ref[...] + b1_ref[...][None, :]
    hm = jnp.maximum(agg, 0.0) * sout_ref[...]
    h2_ref[...] = jnp.dot(hm, w2_ref[...], preferred_element_type=jnp.float32,
                          precision=lax.Precision.HIGHEST)


def _tc_post(p_ref, sin_ref, b2_ref, out_ref):
    p = p_ref[...]
    out_ref[...] = (p[0] + p[1]) * sin_ref[...] + b2_ref[...][None, :]


def kernel(x, edge_index, W1, b1, W2, b2):
    n, d_in = x.shape
    d_hid = W1.shape[1]
    d_out = W2.shape[1]
    e = edge_index.shape[1]

    n_pad = ((n + 1 + 1023) // 1024) * 1024
    n_chunks = -(-e // (_NW * _K))
    e_pad = _NW * _K * n_chunks
    ew = e_pad // _NW

    src = jnp.concatenate([edge_index[0], jnp.full((e_pad - e,), n, jnp.int32)])
    dst = jnp.concatenate([edge_index[1], jnp.full((e_pad - e,), n, jnp.int32)])
    src3 = src.reshape(_NW, n_chunks, _K)
    dst3 = dst.reshape(_NW, n_chunks, _K)
    idx3 = jnp.stack([src, dst]).reshape(2, _NW, ew)

    x_pad = jnp.pad(x, ((0, n_pad - n), (0, 0)))

    degp = _degree_partials(idx3, n_pad)

    f32 = jnp.float32
    h1, sin_col, sout_col = pl.pallas_call(
        _tc_pre,
        out_shape=(
            jax.ShapeDtypeStruct((n_pad, d_hid), f32),
            jax.ShapeDtypeStruct((n_pad, 1), f32),
            jax.ShapeDtypeStruct((n_pad, 1), f32),
        ),
    )(degp, x_pad, W1)

    p1 = _edge_pass(h1, src3, dst3, n_pad, d_hid)

    h2 = pl.pallas_call(
        _tc_mid,
        out_shape=jax.ShapeDtypeStruct((n_pad, d_out), f32),
    )(p1, sin_col, sout_col, b1, W2)

    p2 = _edge_pass(h2, src3, dst3, n_pad, d_out)

    out = pl.pallas_call(
        _tc_post,
        out_shape=jax.ShapeDtypeStruct((n_pad, d_out), f32),
    )(p2, sin_col, b2)

    return out[:n]


# speedup vs baseline: 6.4764x; 6.4764x over previous
import dataclasses
import functools

import jax
import jax.numpy as jnp
from jax import lax
from jax.experimental import pallas as pl
from jax.experimental.pallas import tpu as pltpu
from jax.experimental.pallas import tpu_sc as plsc

_NC = 2
_NS = 16
_NW = _NC * _NS
_LANES = 16
_K = 128
_ZR = 64


def _sc_mesh():
    return plsc.VectorSubcoreMesh(core_axis_name="c", subcore_axis_name="s")


def _sc_params(tc_tiling=True):
    cp = pltpu.CompilerParams()
    fields = pltpu.CompilerParams.__dataclass_fields__
    if "needs_layout_passes" in fields:
        cp = dataclasses.replace(cp, needs_layout_passes=False)
    if not tc_tiling and "use_tc_tiling_on_sc" in fields:
        cp = dataclasses.replace(cp, use_tc_tiling_on_sc=False)
    return cp


def _degree_partials(idx3, n_pad):
    ew = idx3.shape[2]

    def body(idx_hbm, out_hbm, idx_v, hist_v):
        c = lax.axis_index("c")
        s = lax.axis_index("s")
        wid = c * _NS + s
        zeros = jnp.zeros((_LANES,), jnp.float32)
        ones = jnp.ones((_LANES,), jnp.float32)
        for t in range(2):
            @pl.loop(0, n_pad // _LANES)
            def _(i):
                hist_v[pl.ds(i * _LANES, _LANES)] = zeros

            pltpu.sync_copy(idx_hbm.at[t, wid], idx_v)

            @pl.loop(0, ew // _LANES)
            def _(i):
                idx = idx_v[pl.ds(i * _LANES, _LANES)]
                plsc.addupdate_scatter(hist_v, [idx], ones)

            pltpu.sync_copy(hist_v, out_hbm.at[t, wid])

    kern = pl.kernel(
        body,
        out_type=jax.ShapeDtypeStruct((2, _NW, n_pad), jnp.float32),
        mesh=_sc_mesh(),
        compiler_params=_sc_params(),
        scratch_types=[
            pltpu.VMEM((ew,), jnp.int32),
            pltpu.VMEM((n_pad,), jnp.float32),
        ],
    )
    return kern(idx3)


def _edge_pass(h, src3, dst3, n_pad, d):
    n_chunks = src3.shape[1]
    rpw = n_pad // _NS

    def body(h_hbm, src_hbm, dst_hbm, out_hbm, sidx_v, didx_v, msg_v,
             zbuf_v, acc_sh, sem):
        c = lax.axis_index("c")
        s = lax.axis_index("s")
        wid = c * _NS + s
        zeros = jnp.zeros((_LANES,), jnp.float32)

        @pl.loop(0, _ZR)
        def _(r):
            @pl.loop(0, d // _LANES)
            def _(q):
                zbuf_v[r, pl.ds(q * _LANES, _LANES)] = zeros

        @pl.loop(0, rpw // _ZR)
        def _(i):
            pltpu.sync_copy(zbuf_v, acc_sh.at[pl.ds(s * rpw + i * _ZR, _ZR)])

        plsc.subcore_barrier()

        pltpu.sync_copy(src_hbm.at[wid], sidx_v)
        pltpu.sync_copy(dst_hbm.at[wid], didx_v)

        @pl.loop(0, n_chunks)
        def _(j):
            pltpu.async_copy(h_hbm.at[sidx_v.at[j]], msg_v, sem).wait()
            pltpu.sync_copy(msg_v, acc_sh.at[didx_v.at[j]], add=True)

        plsc.subcore_barrier()

        pltpu.sync_copy(acc_sh.at[pl.ds(s * rpw, rpw)],
                        out_hbm.at[c].at[pl.ds(s * rpw, rpw)])

    kern = pl.kernel(
        body,
        out_type=jax.ShapeDtypeStruct((_NC, n_pad, d), jnp.float32),
        mesh=_sc_mesh(),
        compiler_params=_sc_params(tc_tiling=(d % 128 == 0)),
        scratch_types=[
            pltpu.VMEM((n_chunks, _K), jnp.int32),
            pltpu.VMEM((n_chunks, _K), jnp.int32),
            pltpu.VMEM((_K, d), jnp.float32),
            pltpu.VMEM((_ZR, d), jnp.float32),
            pltpu.VMEM_SHARED((n_pad, d), jnp.float32),
            pltpu.SemaphoreType.DMA,
        ],
    )
    return kern(h, src3, dst3)


def _tc_pre(degp_ref, x_ref, w1_ref, h1_ref, sin_ref, sout_ref):
    deg = jnp.sum(degp_ref[...], axis=1)
    sout = lax.rsqrt(jnp.maximum(deg[0], 1.0))
    sin = lax.rsqrt(jnp.maximum(deg[1], 1.0))
    h = x_ref[...] * sout[:, None]
    h1_ref[...] = jnp.dot(h, w1_ref[...], preferred_element_type=jnp.float32,
                          precision=lax.Precision.HIGHEST)
    sin_ref[...] = sin[:, None]
    sout_ref[...] = sout[:, None]


def _tc_mid(p_ref, sin_ref, sout_ref, b1_ref, w2_ref, h2_ref):
    p = p_ref[...]
    agg = (p[0] + p[1]) * sin_ref[...] + b1_ref[...][None, :]
    hm = jnp.maximum(agg, 0.0) * sout_ref[...]
    h2_ref[...] = jnp.dot(hm, w2_ref[...], preferred_element_type=jnp.float32,
                          precision=lax.Precision.HIGHEST)


def _tc_post(p_ref, sin_ref, b2_ref, out_ref):
    p = p_ref[...]
    out_ref[...] = (p[0] + p[1]) * sin_ref[...] + b2_ref[...][None, :]


def kernel(x, edge_index, W1, b1, W2, b2):
    n, d_in = x.shape
    d_hid = W1.shape[1]
    d_out = W2.shape[1]
    e = edge_index.shape[1]

    n_pad = ((n + 1 + 1023) // 1024) * 1024
    n_chunks = -(-e // (_NW * _K))
    e_pad = _NW * _K * n_chunks
    ew = e_pad // _NW

    src = jnp.concatenate([edge_index[0], jnp.full((e_pad - e,), n, jnp.int32)])
    dst = jnp.concatenate([edge_index[1], jnp.full((e_pad - e,), n, jnp.int32)])
    src3 = src.reshape(_NW, n_chunks, _K)
    dst3 = dst.reshape(_NW, n_chunks, _K)
    idx3 = jnp.stack([src, dst]).reshape(2, _NW, ew)

    x_pad = jnp.pad(x, ((0, n_pad - n), (0, 0)))

    degp = _degree_partials(idx3, n_pad)

    f32 = jnp.float32
    h1, sin_col, sout_col = pl.pallas_call(
        _tc_pre,
        out_shape=(
            jax.ShapeDtypeStruct((n_pad, d_hid), f32),
            jax.ShapeDtypeStruct((n_pad, 1), f32),
            jax.ShapeDtypeStruct((n_pad, 1), f32),
        ),
    )(degp, x_pad, W1)

    p1 = _edge_pass(h1, src3, dst3, n_pad, d_hid)

    h2 = pl.pallas_call(
        _tc_mid,
        out_shape=jax.ShapeDtypeStruct((n_pad, d_out), f32),
    )(p1, sin_col, sout_col, b1, W2)

    p2 = _edge_pass(h2, src3, dst3, n_pad, d_out)

    out = pl.pallas_call(
        _tc_post,
        out_shape=jax.ShapeDtypeStruct((n_pad, d_out), f32),
    )(p2, sin_col, b2)

    return out[:n]
